# baseline (speedup 1.0000x reference)
import functools

import jax
import jax.numpy as jnp
from jax import lax
from jax.experimental import pallas as pl
from jax.experimental.pallas import tpu as pltpu
from jax.experimental.pallas import tpu_sc as plsc

B = 16384
L = 50
D = 32
A = 64

NC = 2
NS = 16
NW = NC * NS

PAIR = 2
CHUNK = PAIR * L
ROWS_PER_W = B // NW
CHUNKS_PER_W = ROWS_PER_W // PAIR
NBUF = 4


def _sc_gather_sum(obj2, table):
  mesh = plsc.VectorSubcoreMesh(core_axis_name="c", subcore_axis_name="s")

  @functools.partial(
      pl.kernel,
      out_type=jax.ShapeDtypeStruct((B, D), jnp.float32),
      mesh=mesh,
      compiler_params=pltpu.CompilerParams(use_tc_tiling_on_sc=False),
      scratch_types=[
          pltpu.VMEM((CHUNKS_PER_W, CHUNK), jnp.int32),
          pltpu.VMEM((NBUF, CHUNK, D), jnp.float32),
          pltpu.VMEM((ROWS_PER_W, D), jnp.float32),
      ] + [pltpu.SemaphoreType.DMA] * NBUF,
  )
  def k(obj_hbm, table_hbm, out_hbm, idx_v, gbuf, outbuf, *sems):
    wid = lax.axis_index("s") * NC + lax.axis_index("c")
    cbase = wid * CHUNKS_PER_W

    pltpu.sync_copy(obj_hbm.at[pl.ds(cbase, CHUNKS_PER_W), :], idx_v)

    for slot in range(NBUF):
      pltpu.async_copy(table_hbm.at[idx_v.at[slot]], gbuf.at[slot], sems[slot])

    @pl.loop(0, CHUNKS_PER_W, step=NBUF)
    def _(c0):
      for slot in range(NBUF):
        c = c0 + slot
        pltpu.make_async_copy(
            table_hbm.at[idx_v.at[c]], gbuf.at[slot], sems[slot]).wait()
        for r in range(PAIR):
          base = r * L
          acc = [gbuf[slot, base + k, pl.ds(h * 16, 16)]
                 for k in range(4) for h in range(2)]
          for j in range(4, L):
            k, h = j % 4, 0
            acc[2 * k] = acc[2 * k] + gbuf[slot, base + j, pl.ds(0, 16)]
            acc[2 * k + 1] = acc[2 * k + 1] + gbuf[slot, base + j, pl.ds(16, 16)]
          outbuf[c * PAIR + r, pl.ds(0, 16)] = (
              (acc[0] + acc[2]) + (acc[4] + acc[6]))
          outbuf[c * PAIR + r, pl.ds(16, 16)] = (
              (acc[1] + acc[3]) + (acc[5] + acc[7]))
        nc = c + NBUF
        @pl.when(nc < CHUNKS_PER_W)
        def _():
          pltpu.async_copy(table_hbm.at[idx_v.at[nc]], gbuf.at[slot], sems[slot])

    pltpu.sync_copy(outbuf, out_hbm.at[pl.ds(wid * ROWS_PER_W, ROWS_PER_W), :])

  return k(obj2, table)


def _tc_detile(tableT, vocab):
  nb = 8192

  def tr(in_ref, o_ref, tin_ref):
    x = in_ref[...]
    hi = x.astype(jnp.bfloat16)
    lo = (x - hi.astype(jnp.float32)).astype(jnp.bfloat16)
    stacked = jnp.concatenate([hi, lo], axis=0)
    eye = jnp.eye(D, dtype=jnp.bfloat16)
    eye_sum = jnp.concatenate([eye, eye], axis=0)
    tin_ref[...] = lax.dot_general(
        stacked, eye_sum, (((0,), (0,)), ((), ())),
        preferred_element_type=jnp.float32)
    o_ref[...] = jnp.concatenate(
        [tin_ref[pl.Slice(r, nb // 4, 4), :] for r in range(4)], axis=1)

  return pl.pallas_call(
      tr,
      grid=(pl.cdiv(vocab, nb),),
      in_specs=[pl.BlockSpec((D, nb), lambda i: (0, i))],
      out_specs=pl.BlockSpec((nb // 4, 4 * D), lambda i: (i, 0)),
      out_shape=jax.ShapeDtypeStruct((vocab // 4, 4 * D), jnp.float32),
      scratch_shapes=[pltpu.VMEM((nb, D), jnp.float32)],
  )(tableT)


def _tc_project(vec, v, g, b2):
  bm = 2048

  def mm(vec_ref, v_ref, g_ref, b_ref, o_ref):
    vv = v_ref[...]
    norm = jnp.sqrt(jnp.sum(vv * vv, axis=1, keepdims=True))
    w = (g_ref[...] / (norm * L)) * vv
    o_ref[...] = lax.dot_general(
        w, vec_ref[...], (((1,), (1,)), ((), ())),
        preferred_element_type=jnp.float32) + b_ref[...]

  return pl.pallas_call(
      mm,
      grid=(B // bm,),
      in_specs=[
          pl.BlockSpec((bm, D), lambda i: (i, 0)),
          pl.BlockSpec((A, D), lambda i: (0, 0)),
          pl.BlockSpec((A, 1), lambda i: (0, 0)),
          pl.BlockSpec((A, 1), lambda i: (0, 0)),
      ],
      out_specs=pl.BlockSpec((A, bm), lambda i: (0, i)),
      out_shape=jax.ShapeDtypeStruct((A, B), jnp.float32),
  )(vec, v, g, b2)


def kernel(obj, table, v, g, b):
  obj2 = obj.astype(jnp.int32).reshape(B // PAIR, CHUNK)
  table4 = _tc_detile(table.T, table.shape[0])
  vec_sum = _sc_gather_sum(obj2, table4.reshape(table.shape[0], D))
  return _tc_project(vec_sum, v, g, b.reshape(A, 1)).T


# speedup vs baseline: 1.1204x; 1.1204x over previous
import functools

import jax
import jax.numpy as jnp
from jax import lax
from jax.experimental import pallas as pl
from jax.experimental.pallas import tpu as pltpu
from jax.experimental.pallas import tpu_sc as plsc

B = 16384
L = 50
D = 32
A = 64

NC = 2
NS = 16
NW = NC * NS

PAIR = 2
CHUNK = PAIR * L
ROWS_PER_W = B // NW
CHUNKS_PER_W = ROWS_PER_W // PAIR
NBUF = 4


def _sc_gather_sum(obj2, table):
  mesh = plsc.VectorSubcoreMesh(core_axis_name="c", subcore_axis_name="s")

  @functools.partial(
      pl.kernel,
      out_type=jax.ShapeDtypeStruct((B, D), jnp.float32),
      mesh=mesh,
      compiler_params=pltpu.CompilerParams(use_tc_tiling_on_sc=False),
      scratch_types=[
          pltpu.VMEM((CHUNKS_PER_W, CHUNK), jnp.int32),
          pltpu.VMEM((NBUF, CHUNK, D), jnp.float32),
          pltpu.VMEM((ROWS_PER_W, D), jnp.float32),
      ] + [pltpu.SemaphoreType.DMA] * NBUF,
  )
  def k(obj_hbm, table_hbm, out_hbm, idx_v, gbuf, outbuf, *sems):
    wid = lax.axis_index("s") * NC + lax.axis_index("c")
    cbase = wid * CHUNKS_PER_W

    pltpu.sync_copy(obj_hbm.at[pl.ds(cbase, CHUNKS_PER_W), :], idx_v)

    for slot in range(NBUF):
      pltpu.async_copy(table_hbm.at[idx_v.at[slot]], gbuf.at[slot], sems[slot])

    @pl.loop(0, CHUNKS_PER_W, step=NBUF)
    def _(c0):
      for slot in range(NBUF):
        c = c0 + slot
        pltpu.make_async_copy(
            table_hbm.at[idx_v.at[c]], gbuf.at[slot], sems[slot]).wait()
        for r in range(PAIR):
          base = r * L
          acc = [gbuf[slot, base + k, pl.ds(h * 16, 16)]
                 for k in range(4) for h in range(2)]
          for j in range(4, L):
            k, h = j % 4, 0
            acc[2 * k] = acc[2 * k] + gbuf[slot, base + j, pl.ds(0, 16)]
            acc[2 * k + 1] = acc[2 * k + 1] + gbuf[slot, base + j, pl.ds(16, 16)]
          outbuf[c * PAIR + r, pl.ds(0, 16)] = (
              (acc[0] + acc[2]) + (acc[4] + acc[6]))
          outbuf[c * PAIR + r, pl.ds(16, 16)] = (
              (acc[1] + acc[3]) + (acc[5] + acc[7]))
        nc = c + NBUF
        @pl.when(nc < CHUNKS_PER_W)
        def _():
          pltpu.async_copy(table_hbm.at[idx_v.at[nc]], gbuf.at[slot], sems[slot])

    pltpu.sync_copy(outbuf, out_hbm.at[pl.ds(wid * ROWS_PER_W, ROWS_PER_W), :])

  return k(obj2, table)


def _tc_detile(tableT, vocab):
  nb = 32768

  def tr(in_ref, o_ref, tin_ref):
    x = in_ref[...]
    hi = x.astype(jnp.bfloat16)
    lo = (x - hi.astype(jnp.float32)).astype(jnp.bfloat16)
    stacked = jnp.concatenate([hi, lo], axis=0)
    eye = jnp.eye(D, dtype=jnp.bfloat16)
    eye_sum = jnp.concatenate([eye, eye], axis=0)
    tin_ref[...] = lax.dot_general(
        stacked, eye_sum, (((0,), (0,)), ((), ())),
        preferred_element_type=jnp.float32)
    o_ref[...] = jnp.concatenate(
        [tin_ref[pl.Slice(r, nb // 4, 4), :] for r in range(4)], axis=1)

  return pl.pallas_call(
      tr,
      grid=(pl.cdiv(vocab, nb),),
      in_specs=[pl.BlockSpec((D, nb), lambda i: (0, i))],
      out_specs=pl.BlockSpec((nb // 4, 4 * D), lambda i: (i, 0)),
      out_shape=jax.ShapeDtypeStruct((vocab // 4, 4 * D), jnp.float32),
      scratch_shapes=[pltpu.VMEM((nb, D), jnp.float32)],
  )(tableT)


def _tc_project(vec, v, g, b2):
  bm = 2048

  def mm(vec_ref, v_ref, g_ref, b_ref, o_ref):
    vv = v_ref[...]
    norm = jnp.sqrt(jnp.sum(vv * vv, axis=1, keepdims=True))
    w = (g_ref[...] / (norm * L)) * vv
    o_ref[...] = lax.dot_general(
        w, vec_ref[...], (((1,), (1,)), ((), ())),
        preferred_element_type=jnp.float32) + b_ref[...]

  return pl.pallas_call(
      mm,
      grid=(B // bm,),
      in_specs=[
          pl.BlockSpec((bm, D), lambda i: (i, 0)),
          pl.BlockSpec((A, D), lambda i: (0, 0)),
          pl.BlockSpec((A, 1), lambda i: (0, 0)),
          pl.BlockSpec((A, 1), lambda i: (0, 0)),
      ],
      out_specs=pl.BlockSpec((A, bm), lambda i: (0, i)),
      out_shape=jax.ShapeDtypeStruct((A, B), jnp.float32),
  )(vec, v, g, b2)


def kernel(obj, table, v, g, b):
  obj2 = obj.astype(jnp.int32).reshape(B // PAIR, CHUNK)
  table4 = _tc_detile(table.T, table.shape[0])
  vec_sum = _sc_gather_sum(obj2, table4.reshape(table.shape[0], D))
  return _tc_project(vec_sum, v, g, b.reshape(A, 1)).T
